# baseline (speedup 1.0000x reference)
import jax
import jax.numpy as jnp
from jax.experimental import pallas as pl
from jax.experimental.pallas import tpu as pltpu

_UNROLL = 4


def _cell(xw, h_prev, c_prev, u_ref, H):
    g = xw + jnp.dot(h_prev.astype(jnp.bfloat16), u_ref[...],
                     preferred_element_type=jnp.float32)
    gates = jax.nn.sigmoid(g[:, : 3 * H])
    c_cand = jnp.tanh(g[:, 3 * H :])
    ig = gates[:, :H]
    fg = gates[:, H : 2 * H]
    og = gates[:, 2 * H :]
    c = fg * c_prev + ig * c_cand
    h = og * jnp.tanh(c)
    return h, c


def _lstm_kernel(x_ref, w_ref, u_ref, b_ref, out_ref, h_ref, c_ref, xw_ref):
    H = u_ref.shape[0]
    B = h_ref.shape[0]
    t = pl.program_id(0)

    @pl.when(t <= 1)
    def _():
        h_ref[...] = jnp.zeros_like(h_ref)
        c_ref[...] = jnp.zeros_like(c_ref)

    h = h_ref[...]
    c = c_ref[...]
    xw_all = xw_ref[(t + 1) % 2]
    for k in range(_UNROLL):
        h, c = _cell(xw_all[k * B : (k + 1) * B], h, c, u_ref, H)
        out_ref[k] = h
    c_ref[...] = c
    h_ref[...] = h

    x_all = x_ref[...].reshape(_UNROLL * B, x_ref.shape[2])
    xw_ref[t % 2] = (
        jnp.dot(x_all, w_ref[...], preferred_element_type=jnp.float32)
        + b_ref[...]
    )


def kernel(input_seq, w, u, bias):
    S, B, I = input_seq.shape
    H = u.shape[0]
    U = _UNROLL
    n_chunks = S // U

    x_bf = input_seq.astype(jnp.bfloat16)
    w_bf = w.astype(jnp.bfloat16)
    u_bf = u.astype(jnp.bfloat16)
    b2d = bias.reshape(1, 4 * H)

    out = pl.pallas_call(
        _lstm_kernel,
        out_shape=jax.ShapeDtypeStruct((S, B, H), jnp.float32),
        grid=(n_chunks + 1,),
        in_specs=[
            pl.BlockSpec((U, B, I), lambda t: (jnp.minimum(t, n_chunks - 1), 0, 0)),
            pl.BlockSpec((I, 4 * H), lambda t: (0, 0)),
            pl.BlockSpec((H, 4 * H), lambda t: (0, 0)),
            pl.BlockSpec((1, 4 * H), lambda t: (0, 0)),
        ],
        out_specs=pl.BlockSpec(
            (U, B, H), lambda t: (jnp.maximum(t - 1, 0), 0, 0)
        ),
        scratch_shapes=[
            pltpu.VMEM((B, H), jnp.float32),
            pltpu.VMEM((B, H), jnp.float32),
            pltpu.VMEM((2, U * B, 4 * H), jnp.float32),
        ],
        compiler_params=pltpu.CompilerParams(
            dimension_semantics=("arbitrary",),
            vmem_limit_bytes=56 * 1024 * 1024,
        ),
        name="lstm_fused",
    )(x_bf, w_bf, u_bf, b2d)
    return out


# speedup vs baseline: 1.0130x; 1.0130x over previous
import jax
import jax.numpy as jnp
from jax.experimental import pallas as pl
from jax.experimental.pallas import tpu as pltpu

_UNROLL = 8


def _cell(xw, h_prev, c_prev, u_ref, H):
    g = xw + jnp.dot(h_prev.astype(jnp.bfloat16), u_ref[...],
                     preferred_element_type=jnp.float32)
    gates = jax.nn.sigmoid(g[:, : 3 * H])
    c_cand = jnp.tanh(g[:, 3 * H :])
    ig = gates[:, :H]
    fg = gates[:, H : 2 * H]
    og = gates[:, 2 * H :]
    c = fg * c_prev + ig * c_cand
    h = og * jnp.tanh(c)
    return h, c


def _lstm_step_kernel(x_ref, w_ref, u_ref, b_ref, out_ref, h_ref, c_ref):
    H = u_ref.shape[0]

    @pl.when(pl.program_id(0) == 0)
    def _():
        h_ref[...] = jnp.zeros_like(h_ref)
        c_ref[...] = jnp.zeros_like(c_ref)

    h = h_ref[...]
    c = c_ref[...]

    B = h_ref.shape[0]
    x_all = x_ref[...].reshape(_UNROLL * B, x_ref.shape[2])
    xw_all = (jnp.dot(x_all, w_ref[...], preferred_element_type=jnp.float32)
              + b_ref[...])
    for k in range(_UNROLL):
        h, c = _cell(xw_all[k * B : (k + 1) * B], h, c, u_ref, H)
        out_ref[k] = h

    c_ref[...] = c
    h_ref[...] = h


def kernel(input_seq, w, u, bias):
    S, B, I = input_seq.shape
    H = u.shape[0]

    x_bf = input_seq.astype(jnp.bfloat16)
    w_bf = w.astype(jnp.bfloat16)
    u_bf = u.astype(jnp.bfloat16)
    b2d = bias.reshape(1, 4 * H)

    out = pl.pallas_call(
        _lstm_step_kernel,
        out_shape=jax.ShapeDtypeStruct((S, B, H), jnp.float32),
        grid=(S // _UNROLL,),
        in_specs=[
            pl.BlockSpec((_UNROLL, B, I), lambda t: (t, 0, 0)),
            pl.BlockSpec((I, 4 * H), lambda t: (0, 0)),
            pl.BlockSpec((H, 4 * H), lambda t: (0, 0)),
            pl.BlockSpec((1, 4 * H), lambda t: (0, 0)),
        ],
        out_specs=pl.BlockSpec((_UNROLL, B, H), lambda t: (t, 0, 0)),
        scratch_shapes=[
            pltpu.VMEM((B, H), jnp.float32),
            pltpu.VMEM((B, H), jnp.float32),
        ],
        compiler_params=pltpu.CompilerParams(
            dimension_semantics=("arbitrary",),
            vmem_limit_bytes=56 * 1024 * 1024,
        ),
        name="lstm_fused",
    )(x_bf, w_bf, u_bf, b2d)
    return out


# speedup vs baseline: 1.0151x; 1.0021x over previous
import jax
import jax.numpy as jnp
from jax.experimental import pallas as pl
from jax.experimental.pallas import tpu as pltpu

_UNROLL = 8


def _cell(xw, h_prev, c_prev, u_ref, H):
    g = xw + jnp.dot(h_prev.astype(jnp.bfloat16), u_ref[...],
                     preferred_element_type=jnp.float32)
    gates = 0.5 * jnp.tanh(g[:, : 3 * H] * 0.5) + 0.5
    c_cand = jnp.tanh(g[:, 3 * H :])
    ig = gates[:, :H]
    fg = gates[:, H : 2 * H]
    og = gates[:, 2 * H :]
    c = fg * c_prev + ig * c_cand
    h = og * jnp.tanh(c)
    return h, c


def _lstm_step_kernel(x_ref, w_ref, u_ref, b_ref, out_ref, h_ref, c_ref):
    H = u_ref.shape[0]

    @pl.when(pl.program_id(0) == 0)
    def _():
        h_ref[...] = jnp.zeros_like(h_ref)
        c_ref[...] = jnp.zeros_like(c_ref)

    h = h_ref[...]
    c = c_ref[...]

    B = h_ref.shape[0]
    x_all = x_ref[...].reshape(_UNROLL * B, x_ref.shape[2])
    xw_all = (jnp.dot(x_all, w_ref[...], preferred_element_type=jnp.float32)
              + b_ref[...])
    for k in range(_UNROLL):
        h, c = _cell(xw_all[k * B : (k + 1) * B], h, c, u_ref, H)
        out_ref[k] = h

    c_ref[...] = c
    h_ref[...] = h


def kernel(input_seq, w, u, bias):
    S, B, I = input_seq.shape
    H = u.shape[0]

    x_bf = input_seq.astype(jnp.bfloat16)
    w_bf = w.astype(jnp.bfloat16)
    u_bf = u.astype(jnp.bfloat16)
    b2d = bias.reshape(1, 4 * H)

    out = pl.pallas_call(
        _lstm_step_kernel,
        out_shape=jax.ShapeDtypeStruct((S, B, H), jnp.float32),
        grid=(S // _UNROLL,),
        in_specs=[
            pl.BlockSpec((_UNROLL, B, I), lambda t: (t, 0, 0)),
            pl.BlockSpec((I, 4 * H), lambda t: (0, 0)),
            pl.BlockSpec((H, 4 * H), lambda t: (0, 0)),
            pl.BlockSpec((1, 4 * H), lambda t: (0, 0)),
        ],
        out_specs=pl.BlockSpec((_UNROLL, B, H), lambda t: (t, 0, 0)),
        scratch_shapes=[
            pltpu.VMEM((B, H), jnp.float32),
            pltpu.VMEM((B, H), jnp.float32),
        ],
        compiler_params=pltpu.CompilerParams(
            dimension_semantics=("arbitrary",),
            vmem_limit_bytes=56 * 1024 * 1024,
        ),
        name="lstm_fused",
    )(x_bf, w_bf, u_bf, b2d)
    return out
